# baseline (speedup 1.0000x reference)
import functools
import jax
import jax.numpy as jnp
from jax import lax
from jax.experimental import pallas as pl
from jax.experimental.pallas import tpu as pltpu
from jax.experimental.pallas import tpu_sc as plsc

_R = 8192
_C = 4096
_N = _R * _C
_BR = 512
_NB = _R // _BR

_RT = 6144
_NBT = _RT // _BR
_EOFF = _RT * _C
_NSC = _N - _EOFF
_NW = 32
_PER_W = _NSC // _NW
_CH = 16384
_NPAIR = _PER_W // (2 * _CH)

_mesh = plsc.VectorSubcoreMesh(core_axis_name="c", subcore_axis_name="s")


def _accum_buf(buf, acc_s, acc_q):
    def body(k, carry):
        cs, cq = carry
        base = k * 128
        for j in range(8):
            v = buf[pl.ds(base + j * 16, 16)]
            cs = cs + v
            cq = cq + v * v
        return cs, cq
    return lax.fori_loop(0, _CH // 128, body, (acc_s, acc_q))


@functools.partial(
    pl.kernel,
    mesh=_mesh,
    out_type=[
        jax.ShapeDtypeStruct((_NW, 16), jnp.float32),
        jax.ShapeDtypeStruct((_NW, 16), jnp.float32),
    ],
    scratch_types=[
        pltpu.VMEM((_CH,), jnp.float32),
        pltpu.VMEM((_CH,), jnp.float32),
        pltpu.VMEM((16,), jnp.float32),
        pltpu.VMEM((16,), jnp.float32),
        pltpu.SemaphoreType.DMA,
        pltpu.SemaphoreType.DMA,
    ],
)
def _sc_stats(x_hbm, s_out, q_out, buf0, buf1, st_s, st_q, sem0, sem1):
    wid = lax.axis_index("s") * 2 + lax.axis_index("c")
    base = _EOFF + wid * _PER_W

    pltpu.async_copy(x_hbm.at[pl.ds(base, _CH)], buf0, sem0)

    def pair(j, carry):
        acc_s, acc_q = carry
        c0 = 2 * j
        pltpu.async_copy(x_hbm.at[pl.ds(base + (c0 + 1) * _CH, _CH)], buf1, sem1)
        pltpu.make_async_copy(x_hbm.at[pl.ds(0, _CH)], buf0, sem0).wait()
        acc_s, acc_q = _accum_buf(buf0, acc_s, acc_q)

        @pl.when(j < _NPAIR - 1)
        def _():
            pltpu.async_copy(x_hbm.at[pl.ds(base + (c0 + 2) * _CH, _CH)], buf0, sem0)

        pltpu.make_async_copy(x_hbm.at[pl.ds(0, _CH)], buf1, sem1).wait()
        acc_s, acc_q = _accum_buf(buf1, acc_s, acc_q)
        return acc_s, acc_q

    z = jnp.zeros((16,), jnp.float32)
    acc_s, acc_q = lax.fori_loop(0, _NPAIR, pair, (z, z))
    st_s[...] = acc_s
    st_q[...] = acc_q
    pltpu.sync_copy(st_s, s_out.at[wid])
    pltpu.sync_copy(st_q, q_out.at[wid])


def _tc_stats_kernel(x_ref, o_ref, acc_ref):
    i = pl.program_id(0)
    x = x_ref[...]

    @pl.when(i == 0)
    def _():
        acc_ref[0, 0] = 0.0
        acc_ref[0, 1] = 0.0

    acc_ref[0, 0] += jnp.sum(x)
    acc_ref[0, 1] += jnp.sum(x * x)

    @pl.when(i == _NBT - 1)
    def _():
        o_ref[0, 0] = acc_ref[0, 0]
        o_ref[0, 1] = acc_ref[0, 1]


def _tc_apply_kernel(ab_ref, x_ref, o_ref):
    o_ref[...] = x_ref[...] * ab_ref[0] + ab_ref[1]


def kernel(data, params, scalei, scaleo):
    x2 = data.reshape(_R, _C)
    xf = data.reshape(_N)

    sq_tc = pl.pallas_call(
        _tc_stats_kernel,
        grid=(_NBT,),
        in_specs=[pl.BlockSpec((_BR, _C), lambda i: (i, 0))],
        out_specs=pl.BlockSpec(memory_space=pltpu.SMEM),
        out_shape=jax.ShapeDtypeStruct((1, 2), jnp.float32),
        scratch_shapes=[pltpu.SMEM((1, 2), jnp.float32)],
    )(x2)

    s_p, q_p = _sc_stats(xf)

    s = sq_tc[0, 0] + jnp.sum(s_p)
    q = sq_tc[0, 1] + jnp.sum(q_p)
    mean = s / _N
    var = (q - s * s / _N) / (_N - 1)
    std = jnp.sqrt(var)
    t0 = params[0, 0, 0]
    v0 = params[1, 0, 0]
    ci = scalei.reshape(())
    co = scaleo.reshape(())
    amp = jnp.exp(v0 * jnp.sin(t0)) * ci
    alpha = jnp.sign(amp) * co / std
    beta = -mean * alpha
    ab = jnp.stack([alpha, beta])

    out = pl.pallas_call(
        _tc_apply_kernel,
        grid=(_NB,),
        in_specs=[
            pl.BlockSpec(memory_space=pltpu.SMEM),
            pl.BlockSpec((_BR, _C), lambda i: (i, 0)),
        ],
        out_specs=pl.BlockSpec((_BR, _C), lambda i: (i, 0)),
        out_shape=jax.ShapeDtypeStruct((_R, _C), jnp.float32),
    )(ab, x2)
    return out.reshape(data.shape)


# speedup vs baseline: 1.6160x; 1.6160x over previous
import functools
import jax
import jax.numpy as jnp
from jax import lax
from jax.experimental import pallas as pl
from jax.experimental.pallas import tpu as pltpu
from jax.experimental.pallas import tpu_sc as plsc

_R = 8192
_C = 4096
_N = _R * _C
_BR = 512
_NB = _R // _BR

_RT = 6144
_NBT = _RT // _BR
_NW = 32
_ROWS_W = (_R - _RT) // _NW
_CHR = 4
_NPAIR = _ROWS_W // (2 * _CHR)

_mesh = plsc.VectorSubcoreMesh(core_axis_name="c", subcore_axis_name="s")


def _accum_buf(buf, acc_s, acc_q):
    def body(k, carry):
        cs, cq = carry
        base = k * 128
        for r in range(_CHR):
            for j in range(8):
                v = buf[r, pl.ds(base + j * 16, 16)]
                cs = cs + v
                cq = cq + v * v
        return cs, cq
    return lax.fori_loop(0, _C // 128, body, (acc_s, acc_q))


@functools.partial(
    pl.kernel,
    mesh=_mesh,
    out_type=[
        jax.ShapeDtypeStruct((_NW, 16), jnp.float32),
        jax.ShapeDtypeStruct((_NW, 16), jnp.float32),
    ],
    scratch_types=[
        pltpu.VMEM((_CHR, _C), jnp.float32),
        pltpu.VMEM((_CHR, _C), jnp.float32),
        pltpu.VMEM((16,), jnp.float32),
        pltpu.VMEM((16,), jnp.float32),
        pltpu.SemaphoreType.DMA,
        pltpu.SemaphoreType.DMA,
    ],
)
def _sc_stats(x_hbm, s_out, q_out, buf0, buf1, st_s, st_q, sem0, sem1):
    wid = lax.axis_index("s") * 2 + lax.axis_index("c")
    row0 = _RT + wid * _ROWS_W

    pltpu.async_copy(x_hbm.at[pl.ds(row0, _CHR), :], buf0, sem0)

    def pair(j, carry):
        acc_s, acc_q = carry
        r0 = row0 + 2 * j * _CHR
        pltpu.async_copy(x_hbm.at[pl.ds(r0 + _CHR, _CHR), :], buf1, sem1)
        pltpu.make_async_copy(x_hbm.at[pl.ds(0, _CHR), :], buf0, sem0).wait()
        acc_s, acc_q = _accum_buf(buf0, acc_s, acc_q)

        @pl.when(j < _NPAIR - 1)
        def _():
            pltpu.async_copy(x_hbm.at[pl.ds(r0 + 2 * _CHR, _CHR), :], buf0, sem0)

        pltpu.make_async_copy(x_hbm.at[pl.ds(0, _CHR), :], buf1, sem1).wait()
        acc_s, acc_q = _accum_buf(buf1, acc_s, acc_q)
        return acc_s, acc_q

    z = jnp.zeros((16,), jnp.float32)
    acc_s, acc_q = lax.fori_loop(0, _NPAIR, pair, (z, z))
    st_s[...] = acc_s
    st_q[...] = acc_q
    pltpu.sync_copy(st_s, s_out.at[wid])
    pltpu.sync_copy(st_q, q_out.at[wid])


def _tc_stats_kernel(x_ref, o_ref, acc_ref):
    i = pl.program_id(0)
    x = x_ref[...]

    @pl.when(i == 0)
    def _():
        acc_ref[0, 0] = 0.0
        acc_ref[0, 1] = 0.0

    acc_ref[0, 0] += jnp.sum(x)
    acc_ref[0, 1] += jnp.sum(x * x)

    @pl.when(i == _NBT - 1)
    def _():
        o_ref[0, 0] = acc_ref[0, 0]
        o_ref[0, 1] = acc_ref[0, 1]


def _tc_apply_kernel(ab_ref, x_ref, o_ref):
    o_ref[...] = x_ref[...] * ab_ref[0] + ab_ref[1]


def kernel(data, params, scalei, scaleo):
    x2 = data.reshape(_R, _C)

    sq_tc = pl.pallas_call(
        _tc_stats_kernel,
        grid=(_NBT,),
        in_specs=[pl.BlockSpec((_BR, _C), lambda i: (i, 0))],
        out_specs=pl.BlockSpec(memory_space=pltpu.SMEM),
        out_shape=jax.ShapeDtypeStruct((1, 2), jnp.float32),
        scratch_shapes=[pltpu.SMEM((1, 2), jnp.float32)],
    )(x2)

    s_p, q_p = _sc_stats(x2)

    s = sq_tc[0, 0] + jnp.sum(s_p)
    q = sq_tc[0, 1] + jnp.sum(q_p)
    mean = s / _N
    var = (q - s * s / _N) / (_N - 1)
    std = jnp.sqrt(var)
    t0 = params[0, 0, 0]
    v0 = params[1, 0, 0]
    ci = scalei.reshape(())
    co = scaleo.reshape(())
    amp = jnp.exp(v0 * jnp.sin(t0)) * ci
    alpha = jnp.sign(amp) * co / std
    beta = -mean * alpha
    ab = jnp.stack([alpha, beta])

    out = pl.pallas_call(
        _tc_apply_kernel,
        grid=(_NB,),
        in_specs=[
            pl.BlockSpec(memory_space=pltpu.SMEM),
            pl.BlockSpec((_BR, _C), lambda i: (i, 0)),
        ],
        out_specs=pl.BlockSpec((_BR, _C), lambda i: (i, 0)),
        out_shape=jax.ShapeDtypeStruct((_R, _C), jnp.float32),
    )(ab, x2)
    return out.reshape(data.shape)


# speedup vs baseline: 1.8512x; 1.1455x over previous
import jax
import jax.numpy as jnp
from jax.experimental import pallas as pl
from jax.experimental.pallas import tpu as pltpu

_R = 8192
_C = 4096
_BR = 512
_NB = _R // _BR
_N = _R * _C


def _fused_kernel(sc_ref, x_ref, o_ref, acc_ref):
    p = pl.program_id(0)
    i = pl.program_id(1)

    @pl.when(p == 0)
    def _reduce():
        x = x_ref[...]
        s = jnp.sum(x)
        q = jnp.sum(x * x)

        @pl.when(i == 0)
        def _():
            acc_ref[0, 0] = 0.0
            acc_ref[0, 1] = 0.0

        acc_ref[0, 0] += s
        acc_ref[0, 1] += q

    @pl.when(p == 1)
    def _apply():
        @pl.when(i == 0)
        def _():
            s = acc_ref[0, 0]
            q = acc_ref[0, 1]
            mean = s / _N
            var = (q - s * s / _N) / (_N - 1)
            std = jnp.sqrt(var)
            t0 = sc_ref[0]
            v0 = sc_ref[1]
            ci = sc_ref[2]
            co = sc_ref[3]
            amp = jnp.exp(v0 * jnp.sin(t0)) * ci
            alpha = jnp.sign(amp) * co / std
            acc_ref[0, 2] = alpha
            acc_ref[0, 3] = -mean * alpha

        alpha = acc_ref[0, 2]
        beta = acc_ref[0, 3]
        o_ref[...] = x_ref[...] * alpha + beta


def kernel(data, params, scalei, scaleo):
    x = data.reshape(_R, _C)
    scalars = jnp.stack([
        params[0, 0, 0],
        params[1, 0, 0],
        scalei.reshape(()),
        scaleo.reshape(()),
    ])
    out = pl.pallas_call(
        _fused_kernel,
        grid=(2, _NB),
        in_specs=[
            pl.BlockSpec(memory_space=pltpu.SMEM),
            pl.BlockSpec((_BR, _C), lambda p, i: (i, 0)),
        ],
        out_specs=pl.BlockSpec((_BR, _C), lambda p, i: (i * p, 0)),
        out_shape=jax.ShapeDtypeStruct((_R, _C), jnp.float32),
        scratch_shapes=[pltpu.SMEM((1, 4), jnp.float32)],
    )(scalars, x)
    return out.reshape(data.shape)


# speedup vs baseline: 1.8865x; 1.0191x over previous
import jax
import jax.numpy as jnp
from jax.experimental import pallas as pl
from jax.experimental.pallas import tpu as pltpu

_R = 8192
_C = 4096
_BR = 512
_NB = _R // _BR
_N = _R * _C
_NCACHE = 2


def _fused_kernel(sc_ref, x_ref, o_ref, acc_ref, cache_ref):
    p = pl.program_id(0)
    i = pl.program_id(1)

    @pl.when(p == 0)
    def _reduce():
        x = x_ref[...]
        s = jnp.sum(x)
        q = jnp.sum(x * x)

        @pl.when(i == 0)
        def _():
            acc_ref[0, 0] = 0.0
            acc_ref[0, 1] = 0.0

        acc_ref[0, 0] += s
        acc_ref[0, 1] += q

        for c in range(_NCACHE):
            @pl.when(i == c)
            def _(c=c):
                cache_ref[c] = x

    @pl.when(p == 1)
    def _apply():
        @pl.when(i == 0)
        def _():
            s = acc_ref[0, 0]
            q = acc_ref[0, 1]
            mean = s / _N
            var = (q - s * s / _N) / (_N - 1)
            std = jnp.sqrt(var)
            t0 = sc_ref[0]
            v0 = sc_ref[1]
            ci = sc_ref[2]
            co = sc_ref[3]
            amp = jnp.exp(v0 * jnp.sin(t0)) * ci
            alpha = jnp.sign(amp) * co / std
            acc_ref[0, 2] = alpha
            acc_ref[0, 3] = -mean * alpha

        alpha = acc_ref[0, 2]
        beta = acc_ref[0, 3]

        @pl.when(i >= _NCACHE)
        def _():
            o_ref[...] = x_ref[...] * alpha + beta

        for c in range(_NCACHE):
            @pl.when(i == c)
            def _(c=c):
                o_ref[...] = cache_ref[c] * alpha + beta


def _in_index(p, i):
    return (jnp.where(p == 0, i, jnp.maximum(i, _NCACHE)), 0)


def kernel(data, params, scalei, scaleo):
    x = data.reshape(_R, _C)
    scalars = jnp.stack([
        params[0, 0, 0],
        params[1, 0, 0],
        scalei.reshape(()),
        scaleo.reshape(()),
    ])
    out = pl.pallas_call(
        _fused_kernel,
        grid=(2, _NB),
        in_specs=[
            pl.BlockSpec(memory_space=pltpu.SMEM),
            pl.BlockSpec((_BR, _C), _in_index),
        ],
        out_specs=pl.BlockSpec((_BR, _C), lambda p, i: (i * p, 0)),
        out_shape=jax.ShapeDtypeStruct((_R, _C), jnp.float32),
        scratch_shapes=[
            pltpu.SMEM((1, 4), jnp.float32),
            pltpu.VMEM((_NCACHE, _BR, _C), jnp.float32),
        ],
    )(scalars, x)
    return out.reshape(data.shape)
